# Initial kernel scaffold; baseline (speedup 1.0000x reference)
import jax
import jax.numpy as jnp
from jax.experimental import pallas as pl


def kernel(x, emb_table, W1, b1, W2, b2):
    raise NotImplementedError("write your pallas kernel here")


# speedup vs baseline: 1.3930x; 1.3930x over previous
import functools

import jax
import jax.numpy as jnp
from jax import lax
from jax.experimental import pallas as pl
from jax.experimental.pallas import tpu as pltpu
from jax.experimental.pallas import tpu_sc as plsc

_BUCKETS = 100
_DEMB = 64
_N = 4096 * 200
_S = _N * 2
_LANES_TC = 256
_ROWS_TC = _S // _LANES_TC
_BLK_TC = 256

_NW = 32
_PER_W = _S // _NW
_CHUNK = 1024
_NCHUNK = _PER_W // _CHUNK


def _tc_weights_body(x_ref, w1_ref, b1_ref, w2_ref, b2_ref, w_ref, i_ref):
    xv = x_ref[...]
    xn = pltpu.roll(xv, _LANES_TC - 1, 1)
    d = jnp.full(xv.shape, b2_ref[1, 0] - b2_ref[0, 0], jnp.float32)
    for o in range(32):
        h = xv * w1_ref[o, 0] + xn * w1_ref[o, 1] + b1_ref[o, 0]
        h = jnp.maximum(h, h * 0.01)
        d = d + (w2_ref[1, o] - w2_ref[0, o]) * h
    we = 1.0 / (1.0 + jnp.exp(d))
    lane = lax.broadcasted_iota(jnp.int32, xv.shape, 1)
    even = (lane % 2) == 0
    w_ref[...] = jnp.where(even, we, 1.0 - pltpu.roll(we, 1, 1))
    idx = jnp.floor(xv * jnp.float32(_BUCKETS)).astype(jnp.int32)
    i_ref[...] = jnp.clip(idx, 0, _BUCKETS - 1)


def _tc_weights(xr, W1, b1, W2, b2):
    grid = (_ROWS_TC // _BLK_TC,)
    blk = pl.BlockSpec((_BLK_TC, _LANES_TC), lambda i: (i, 0))
    rep2 = lambda shape: pl.BlockSpec(shape, lambda i: (0, 0))
    return pl.pallas_call(
        _tc_weights_body,
        grid=grid,
        in_specs=[blk, rep2((32, 2)), rep2((32, 1)), rep2((2, 32)), rep2((2, 1))],
        out_specs=[blk, blk],
        out_shape=[
            jax.ShapeDtypeStruct((_ROWS_TC, _LANES_TC), jnp.float32),
            jax.ShapeDtypeStruct((_ROWS_TC, _LANES_TC), jnp.int32),
        ],
    )(xr, W1, b1.reshape(32, 1), W2, b2.reshape(2, 1))


def _sc_body(idx_hbm, w_hbm, tab_hbm, out_hbm, tab_v, idx_v, w_v, out_v):
    wid = lax.axis_index("c") * 16 + lax.axis_index("s")
    base = wid * _PER_W
    pltpu.sync_copy(tab_hbm, tab_v)
    lane64 = lax.broadcasted_iota(jnp.int32, (16,), 0) * _DEMB

    def chunk(ci, carry):
        cb = base + ci * _CHUNK
        pltpu.sync_copy(idx_hbm.at[pl.ds(cb, _CHUNK)], idx_v)
        pltpu.sync_copy(w_hbm.at[pl.ds(cb, _CHUNK)], w_v)

        def group(g, c2):
            iv = idx_v[pl.ds(g * 16, 16)]
            wv = w_v[pl.ds(g * 16, 16)]
            a0 = iv * _DEMB
            sb = lane64 + g * (16 * _DEMB)
            for c in range(_DEMB):
                tv = plsc.load_gather(tab_v, [a0 + c])
                plsc.store_scatter(out_v, [sb + c], tv * wv)
            return c2

        lax.fori_loop(0, _CHUNK // 16, group, 0)
        pltpu.sync_copy(out_v, out_hbm.at[pl.ds(cb * _DEMB, _CHUNK * _DEMB)])
        return carry

    lax.fori_loop(0, _NCHUNK, chunk, 0)


def _sc_gather(idx_flat, w_flat, tab_flat):
    mesh = plsc.VectorSubcoreMesh(core_axis_name="c", subcore_axis_name="s")
    k = functools.partial(
        pl.kernel,
        mesh=mesh,
        compiler_params=pltpu.CompilerParams(needs_layout_passes=False),
        out_type=jax.ShapeDtypeStruct((_S * _DEMB,), jnp.float32),
        scratch_types=[
            pltpu.VMEM((_BUCKETS * _DEMB,), jnp.float32),
            pltpu.VMEM((_CHUNK,), jnp.int32),
            pltpu.VMEM((_CHUNK,), jnp.float32),
            pltpu.VMEM((_CHUNK * _DEMB,), jnp.float32),
        ],
    )(_sc_body)
    return k(idx_flat, w_flat, tab_flat)


def kernel(x, emb_table, W1, b1, W2, b2):
    xr = x.reshape(_ROWS_TC, _LANES_TC)
    w, i = _tc_weights(xr, W1, b1, W2, b2)
    out = _sc_gather(i.reshape(_S), w.reshape(_S), emb_table.reshape(_BUCKETS * _DEMB))
    return out.reshape(4096, 200, 128)


# speedup vs baseline: 4.7818x; 3.4327x over previous
import functools

import jax
import jax.numpy as jnp
from jax import lax
from jax.experimental import pallas as pl
from jax.experimental.pallas import tpu as pltpu
from jax.experimental.pallas import tpu_sc as plsc

_BUCKETS = 100
_DEMB = 64
_PAD = 79
_N = 4096 * 200
_S = _N * 2
_LANES_TC = 256
_ROWS_TC = _S // _LANES_TC
_BLK_TC = 256

_NW = 32
_PER_W = _S // _NW
_CHUNK = 512
_CWORDS = _CHUNK * _DEMB
_SUPER = 10
_NSUPER = _PER_W // (_SUPER * _CHUNK)


def _tc_weights_body(x_ref, w1_ref, b1_ref, w2_ref, b2_ref, w_ref, i_ref):
    xv = x_ref[...]
    xn = pltpu.roll(xv, _LANES_TC - 1, 1)
    d = jnp.full(xv.shape, b2_ref[1, 0] - b2_ref[0, 0], jnp.float32)
    for o in range(32):
        h = xv * w1_ref[o, 0] + xn * w1_ref[o, 1] + b1_ref[o, 0]
        h = jnp.maximum(h, h * 0.01)
        d = d + (w2_ref[1, o] - w2_ref[0, o]) * h
    we = 1.0 / (1.0 + jnp.exp(d))
    lane = lax.broadcasted_iota(jnp.int32, xv.shape, 1)
    even = (lane % 2) == 0
    w_ref[...] = jnp.where(even, we, 1.0 - pltpu.roll(we, 1, 1))
    idx = jnp.floor(xv * jnp.float32(_BUCKETS)).astype(jnp.int32)
    i_ref[...] = jnp.clip(idx, 0, _BUCKETS - 1)


def _tc_weights(xr, W1, b1, W2, b2):
    grid = (_ROWS_TC // _BLK_TC,)
    blk = pl.BlockSpec((_BLK_TC, _LANES_TC), lambda i: (i, 0))
    rep2 = lambda shape: pl.BlockSpec(shape, lambda i: (0, 0))
    return pl.pallas_call(
        _tc_weights_body,
        grid=grid,
        in_specs=[blk, rep2((32, 2)), rep2((32, 1)), rep2((2, 32)), rep2((2, 1))],
        out_specs=[blk, blk],
        out_shape=[
            jax.ShapeDtypeStruct((_ROWS_TC, _LANES_TC), jnp.float32),
            jax.ShapeDtypeStruct((_ROWS_TC, _LANES_TC), jnp.int32),
        ],
    )(xr, W1, b1.reshape(32, 1), W2, b2.reshape(2, 1))


def _sc_body(idx_hbm, w_hbm, tab_hbm, out_hbm,
             tab_v, idx_v, w_v, out_v, sem0, sem1):
    wid = lax.axis_index("c") * 16 + lax.axis_index("s")
    base = wid * _PER_W
    pltpu.sync_copy(tab_hbm, tab_v)
    iota = lax.broadcasted_iota(jnp.int32, (16,), 0)
    lane64 = iota * _DEMB

    kvec = [lane64 + ((iota + c) & (_DEMB - 1)) for c in range(_DEMB)]
    tabw = _BUCKETS * _PAD

    def fill(local_base, buf):
        boff = buf * _CWORDS

        def group(g, c2):
            iv = idx_v[pl.ds(local_base + g * 16, 16)]
            wv = w_v[pl.ds(local_base + g * 16, 16)]
            a79 = iv * _PAD + iota
            sg = boff + g * (16 * _DEMB)
            for cw in range(0, _DEMB, 8):
                tvs = [
                    plsc.load_gather(tab_v.at[pl.ds(cw, tabw - cw)], [a79 + j])
                    for j in range(8)
                ]
                rs = [tv * wv for tv in tvs]
                for j in range(8):
                    plsc.store_scatter(out_v, [kvec[cw + j] + sg], rs[j])
            return c2

        lax.fori_loop(0, _CHUNK // 16, group, 0)

    def superblock(sp, carry):
        sb_base = base + sp * (_SUPER * _CHUNK)
        pltpu.sync_copy(idx_hbm.at[pl.ds(sb_base, _SUPER * _CHUNK)], idx_v)
        pltpu.sync_copy(w_hbm.at[pl.ds(sb_base, _SUPER * _CHUNK)], w_v)

        def pair(pj, c2):
            for b in range(2):
                ci = pj * 2 + b
                gci = sp * _SUPER + ci

                @pl.when(gci >= 2)
                def _wait():
                    sem = sem0 if b == 0 else sem1
                    pltpu.make_async_copy(
                        out_v.at[pl.ds(b * _CWORDS, _CWORDS)],
                        out_hbm.at[pl.ds(0, _CWORDS)], sem).wait()

                fill(ci * _CHUNK, b)
                pltpu.async_copy(
                    out_v.at[pl.ds(b * _CWORDS, _CWORDS)],
                    out_hbm.at[pl.ds((sb_base + ci * _CHUNK) * _DEMB, _CWORDS)],
                    sem0 if b == 0 else sem1)
            return c2

        lax.fori_loop(0, _SUPER // 2, pair, 0)
        return carry

    lax.fori_loop(0, _NSUPER, superblock, 0)
    for b in range(2):
        pltpu.make_async_copy(
            out_v.at[pl.ds(b * _CWORDS, _CWORDS)],
            out_hbm.at[pl.ds(0, _CWORDS)],
            sem0 if b == 0 else sem1).wait()


def _sc_gather(idx_flat, w_flat, tab_flat):
    mesh = plsc.VectorSubcoreMesh(core_axis_name="c", subcore_axis_name="s")
    k = functools.partial(
        pl.kernel,
        mesh=mesh,
        compiler_params=pltpu.CompilerParams(needs_layout_passes=False),
        out_type=jax.ShapeDtypeStruct((_S * _DEMB,), jnp.float32),
        scratch_types=[
            pltpu.VMEM((_BUCKETS * _PAD,), jnp.float32),
            pltpu.VMEM((_SUPER * _CHUNK,), jnp.int32),
            pltpu.VMEM((_SUPER * _CHUNK,), jnp.float32),
            pltpu.VMEM((2 * _CWORDS,), jnp.float32),
            pltpu.SemaphoreType.DMA,
            pltpu.SemaphoreType.DMA,
        ],
    )(_sc_body)
    return k(idx_flat, w_flat, tab_flat)


def kernel(x, emb_table, W1, b1, W2, b2):
    xr = x.reshape(_ROWS_TC, _LANES_TC)
    w, i = _tc_weights(xr, W1, b1, W2, b2)
    tab79 = jnp.concatenate([emb_table, emb_table[:, : _PAD - _DEMB]], axis=1)
    out = _sc_gather(i.reshape(_S), w.reshape(_S), tab79.reshape(_BUCKETS * _PAD))
    return out.reshape(4096, 200, 128)


# speedup vs baseline: 4.9144x; 1.0277x over previous
import functools

import jax
import jax.numpy as jnp
from jax import lax
from jax.experimental import pallas as pl
from jax.experimental.pallas import tpu as pltpu
from jax.experimental.pallas import tpu_sc as plsc

_BUCKETS = 100
_DEMB = 64
_PAD = 79
_N = 4096 * 200
_S = _N * 2
_LANES_TC = 256
_ROWS_TC = _S // _LANES_TC
_BLK_TC = 256

_NW = 32
_PER_W = _S // _NW
_CHUNK = 512
_CROWS = _CHUNK // 2
_SUPER = 20
_SROWS = _SUPER * _CHUNK // _LANES_TC
_NSUPER = _PER_W // (_SUPER * _CHUNK)


def _tc_weights_body(x_ref, w1_ref, b1_ref, w2_ref, b2_ref, w_ref, i_ref):
    xv = x_ref[...]
    xn = pltpu.roll(xv, _LANES_TC - 1, 1)
    d = jnp.full(xv.shape, b2_ref[1, 0] - b2_ref[0, 0], jnp.float32)
    for o in range(32):
        h = xv * w1_ref[o, 0] + xn * w1_ref[o, 1] + b1_ref[o, 0]
        h = jnp.maximum(h, h * 0.01)
        d = d + (w2_ref[1, o] - w2_ref[0, o]) * h
    we = 1.0 / (1.0 + jnp.exp(d))
    lane = lax.broadcasted_iota(jnp.int32, xv.shape, 1)
    even = (lane % 2) == 0
    w_ref[...] = jnp.where(even, we, 1.0 - pltpu.roll(we, 1, 1))
    idx = jnp.floor(xv * jnp.float32(_BUCKETS)).astype(jnp.int32)
    i_ref[...] = jnp.clip(idx, 0, _BUCKETS - 1)


def _tc_weights(xr, W1, b1, W2, b2):
    grid = (_ROWS_TC // _BLK_TC,)
    blk = pl.BlockSpec((_BLK_TC, _LANES_TC), lambda i: (i, 0))
    rep2 = lambda shape: pl.BlockSpec(shape, lambda i: (0, 0))
    return pl.pallas_call(
        _tc_weights_body,
        grid=grid,
        in_specs=[blk, rep2((32, 2)), rep2((32, 1)), rep2((2, 32)), rep2((2, 1))],
        out_specs=[blk, blk],
        out_shape=[
            jax.ShapeDtypeStruct((_ROWS_TC, _LANES_TC), jnp.float32),
            jax.ShapeDtypeStruct((_ROWS_TC, _LANES_TC), jnp.int32),
        ],
    )(xr, W1, b1.reshape(32, 1), W2, b2.reshape(2, 1))


def _sc_body(idx_hbm, w_hbm, tab_hbm, out_hbm,
             tab_v, idx_v, w_v, out_v, sem0, sem1):
    wid = lax.axis_index("c") * 16 + lax.axis_index("s")
    base_row = wid * (_PER_W // _LANES_TC)
    out_row_base = wid * (_PER_W // 2)
    pltpu.sync_copy(tab_hbm, tab_v)
    iota = lax.broadcasted_iota(jnp.int32, (16,), 0)
    kcol = [(iota & 1) * _DEMB + ((iota + c) & (_DEMB - 1))
            for c in range(_DEMB)]
    rowi = iota // 2
    tabw = _BUCKETS * _PAD

    def fill(local_sub, buf):
        brow = buf * _CROWS

        def group(g, c2):
            s0 = local_sub + g * 16
            gr = s0 // _LANES_TC
            gl = s0 % _LANES_TC
            iv = idx_v[gr, pl.ds(gl, 16)]
            wv = w_v[gr, pl.ds(gl, 16)]
            a79 = iv * _PAD + iota
            rowv = brow + g * 8 + rowi
            for cw in range(0, _DEMB, 8):
                tvs = [
                    plsc.load_gather(tab_v.at[pl.ds(cw, tabw - cw)], [a79 + j])
                    for j in range(8)
                ]
                rs = [tv * wv for tv in tvs]
                for j in range(8):
                    plsc.store_scatter(out_v, [rowv, kcol[cw + j]], rs[j])
            return c2

        lax.fori_loop(0, _CHUNK // 16, group, 0)

    def superblock(sp, carry):
        srow = base_row + sp * _SROWS
        pltpu.sync_copy(idx_hbm.at[pl.ds(srow, _SROWS), :], idx_v)
        pltpu.sync_copy(w_hbm.at[pl.ds(srow, _SROWS), :], w_v)

        def pair(pj, c2):
            for b in range(2):
                ci = pj * 2 + b
                gci = sp * _SUPER + ci

                @pl.when(gci >= 2)
                def _wait():
                    sem = sem0 if b == 0 else sem1
                    pltpu.make_async_copy(
                        out_v.at[pl.ds(b * _CROWS, _CROWS), :],
                        out_hbm.at[pl.ds(0, _CROWS), :], sem).wait()

                fill(ci * _CHUNK, b)
                dst_row = out_row_base + (sp * _SUPER + ci) * _CROWS
                pltpu.async_copy(
                    out_v.at[pl.ds(b * _CROWS, _CROWS), :],
                    out_hbm.at[pl.ds(dst_row, _CROWS), :],
                    sem0 if b == 0 else sem1)
            return c2

        lax.fori_loop(0, _SUPER // 2, pair, 0)
        return carry

    lax.fori_loop(0, _NSUPER, superblock, 0)
    for b in range(2):
        pltpu.make_async_copy(
            out_v.at[pl.ds(b * _CROWS, _CROWS), :],
            out_hbm.at[pl.ds(0, _CROWS), :],
            sem0 if b == 0 else sem1).wait()


def _sc_gather(idx2, w2, tab_flat):
    mesh = plsc.VectorSubcoreMesh(core_axis_name="c", subcore_axis_name="s")
    k = functools.partial(
        pl.kernel,
        mesh=mesh,
        compiler_params=pltpu.CompilerParams(needs_layout_passes=False),
        out_type=jax.ShapeDtypeStruct((_N, 128), jnp.float32),
        scratch_types=[
            pltpu.VMEM((_BUCKETS * _PAD,), jnp.float32),
            pltpu.VMEM((_SROWS, _LANES_TC), jnp.int32),
            pltpu.VMEM((_SROWS, _LANES_TC), jnp.float32),
            pltpu.VMEM((2 * _CROWS, 128), jnp.float32),
            pltpu.SemaphoreType.DMA,
            pltpu.SemaphoreType.DMA,
        ],
    )(_sc_body)
    return k(idx2, w2, tab_flat)


def kernel(x, emb_table, W1, b1, W2, b2):
    xr = x.reshape(_ROWS_TC, _LANES_TC)
    w, i = _tc_weights(xr, W1, b1, W2, b2)
    tab79 = jnp.concatenate([emb_table, emb_table[:, : _PAD - _DEMB]], axis=1)
    out = _sc_gather(i, w, tab79.reshape(_BUCKETS * _PAD))
    return out.reshape(4096, 200, 128)


# speedup vs baseline: 4.9166x; 1.0004x over previous
import functools

import jax
import jax.numpy as jnp
from jax import lax
from jax.experimental import pallas as pl
from jax.experimental.pallas import tpu as pltpu
from jax.experimental.pallas import tpu_sc as plsc

_BUCKETS = 100
_DEMB = 64
_PAD = 79
_N = 4096 * 200
_S = _N * 2
_LANES_TC = 256
_ROWS_TC = _S // _LANES_TC
_BLK_TC = 256

_NW = 32
_PER_W = _S // _NW
_CHUNK = 512
_CROWS = _CHUNK // 2
_SUPER = 20
_SROWS = _SUPER * _CHUNK // _LANES_TC
_NSUPER = _PER_W // (_SUPER * _CHUNK)


def _tc_weights_body(x_ref, w1_ref, b1_ref, w2_ref, b2_ref, w_ref, i_ref):
    xv = x_ref[...]
    xn = pltpu.roll(xv, _LANES_TC - 1, 1)
    d = jnp.full(xv.shape, b2_ref[1, 0] - b2_ref[0, 0], jnp.float32)
    for o in range(32):
        h = xv * w1_ref[o, 0] + xn * w1_ref[o, 1] + b1_ref[o, 0]
        h = jnp.maximum(h, h * 0.01)
        d = d + (w2_ref[1, o] - w2_ref[0, o]) * h
    we = 1.0 / (1.0 + jnp.exp(d))
    lane = lax.broadcasted_iota(jnp.int32, xv.shape, 1)
    even = (lane % 2) == 0
    w_ref[...] = jnp.where(even, we, 1.0 - pltpu.roll(we, 1, 1))
    idx = jnp.floor(xv * jnp.float32(_BUCKETS)).astype(jnp.int32)
    i_ref[...] = jnp.clip(idx, 0, _BUCKETS - 1)


def _tc_weights(xr, W1, b1, W2, b2):
    grid = (_ROWS_TC // _BLK_TC,)
    blk = pl.BlockSpec((_BLK_TC, _LANES_TC), lambda i: (i, 0))
    rep2 = lambda shape: pl.BlockSpec(shape, lambda i: (0, 0))
    return pl.pallas_call(
        _tc_weights_body,
        grid=grid,
        in_specs=[blk, rep2((32, 2)), rep2((32, 1)), rep2((2, 32)), rep2((2, 1))],
        out_specs=[blk, blk],
        out_shape=[
            jax.ShapeDtypeStruct((_ROWS_TC, _LANES_TC), jnp.float32),
            jax.ShapeDtypeStruct((_ROWS_TC, _LANES_TC), jnp.int32),
        ],
    )(xr, W1, b1.reshape(32, 1), W2, b2.reshape(2, 1))


def _sc_body(idx_hbm, w_hbm, tab_hbm, out_hbm,
             tab_v, idx_v, w_v, out_v, sem0, sem1):
    wid = lax.axis_index("c") * 16 + lax.axis_index("s")
    base_row = wid * (_PER_W // _LANES_TC)
    out_row_base = wid * (_PER_W // 2)
    pltpu.sync_copy(tab_hbm, tab_v)
    iota = lax.broadcasted_iota(jnp.int32, (16,), 0)
    kcol = [(iota & 1) * _DEMB + ((iota + c) & (_DEMB - 1))
            for c in range(_DEMB)]
    rowi = iota // 2
    tabw = _BUCKETS * _PAD

    def fill(local_sub, buf):
        brow = buf * _CROWS

        def group(g, c2):
            s0 = local_sub + g * 16
            gr = s0 // _LANES_TC
            gl = s0 % _LANES_TC
            iv = idx_v[gr, pl.ds(gl, 16)]
            wv = w_v[gr, pl.ds(gl, 16)]
            a79 = iv * _PAD + iota
            rowv = brow + g * 8 + rowi
            for cw in range(0, _DEMB, 8):
                tvs = [
                    plsc.load_gather(tab_v.at[pl.ds(cw, tabw - cw)], [a79 + j])
                    for j in range(8)
                ]
                rs = [tv * wv for tv in tvs]
                for j in range(8):
                    plsc.store_scatter(out_v, [rowv, kcol[cw + j]], rs[j])
            return c2

        lax.fori_loop(0, _CHUNK // 16, group, 0)

    def superblock(sp, carry):
        srow = base_row + sp * _SROWS
        pltpu.sync_copy(idx_hbm.at[pl.ds(srow, _SROWS), :], idx_v)
        pltpu.sync_copy(w_hbm.at[pl.ds(srow, _SROWS), :], w_v)

        def pair(pj, c2):
            for b in range(2):
                ci = pj * 2 + b
                gci = sp * _SUPER + ci

                @pl.when(gci >= 2)
                def _wait():
                    sem = sem0 if b == 0 else sem1
                    pltpu.make_async_copy(
                        out_v.at[pl.ds(b * _CROWS, _CROWS), :],
                        out_hbm.at[pl.ds(0, _CROWS), :], sem).wait()

                fill(ci * _CHUNK, b)
                dst_row = out_row_base + (sp * _SUPER + ci) * _CROWS
                pltpu.async_copy(
                    out_v.at[pl.ds(b * _CROWS, _CROWS), :],
                    out_hbm.at[pl.ds(dst_row, _CROWS), :],
                    sem0 if b == 0 else sem1)
            return c2

        lax.fori_loop(0, _SUPER // 2, pair, 0)
        return carry

    lax.fori_loop(0, _NSUPER, superblock, 0)
    for b in range(2):
        pltpu.make_async_copy(
            out_v.at[pl.ds(b * _CROWS, _CROWS), :],
            out_hbm.at[pl.ds(0, _CROWS), :],
            sem0 if b == 0 else sem1).wait()


def _sc_gather(idx2, w2, tab_flat):
    mesh = plsc.VectorSubcoreMesh(core_axis_name="c", subcore_axis_name="s")
    k = functools.partial(
        pl.kernel,
        mesh=mesh,
        compiler_params=pltpu.CompilerParams(
            needs_layout_passes=False, use_tc_tiling_on_sc=True),
        out_type=jax.ShapeDtypeStruct((_N, 128), jnp.float32),
        scratch_types=[
            pltpu.VMEM((_BUCKETS * _PAD,), jnp.float32),
            pltpu.VMEM((_SROWS, _LANES_TC), jnp.int32),
            pltpu.VMEM((_SROWS, _LANES_TC), jnp.float32),
            pltpu.VMEM((2 * _CROWS, 128), jnp.float32),
            pltpu.SemaphoreType.DMA,
            pltpu.SemaphoreType.DMA,
        ],
    )(_sc_body)
    return k(idx2, w2, tab_flat)


def kernel(x, emb_table, W1, b1, W2, b2):
    xr = x.reshape(_ROWS_TC, _LANES_TC)
    w, i = _tc_weights(xr, W1, b1, W2, b2)
    tab79 = jnp.concatenate([emb_table, emb_table[:, : _PAD - _DEMB]], axis=1)
    out = _sc_gather(i, w, tab79.reshape(_BUCKETS * _PAD))
    return out.reshape(4096, 200, 128)
